# baseline (speedup 1.0000x reference)
import functools

import jax
import jax.numpy as jnp
from jax.experimental import pallas as pl
from jax.experimental.pallas import tpu as pltpu

_MT = 1024
_CK = 1024


def _rvq_body(audio_ref, sem_ref, w_in_ref, b_in_ref, cb_ref, w_sem_ref,
              b_sem_ref, quant_ref, idx_ref, loss_ref, x_ref, res_ref):
    i = pl.program_id(0)
    m = audio_ref.shape[0]
    d = w_in_ref.shape[1]
    k = cb_ref.shape[1]
    b = sem_ref.shape[0]
    s = m // b
    n_sem = w_sem_ref.shape[0]

    @pl.when(i == 0)
    def _init():
        x = jnp.dot(audio_ref[...], w_in_ref[...],
                    preferred_element_type=jnp.float32) + b_in_ref[...]
        x_ref[...] = x
        res_ref[...] = x
        quant_ref[...] = jnp.zeros_like(quant_ref)
        loss_ref[0, 0] = 0.0

    @pl.when(i < n_sem)
    def _semantic_bias():
        j = jnp.minimum(i, n_sem - 1)
        bias = jnp.dot(sem_ref[...], w_sem_ref[j],
                       preferred_element_type=jnp.float32) + b_sem_ref[j]
        rows = jax.lax.broadcasted_iota(jnp.int32, (m, b), 0) // s
        cols = jax.lax.broadcasted_iota(jnp.int32, (m, b), 1)
        sel = (rows == cols).astype(jnp.float32)
        res_ref[...] = res_ref[...] + 0.1 * jnp.dot(
            sel, bias, preferred_element_type=jnp.float32,
            precision=jax.lax.Precision.HIGHEST)

    nck = k // _CK
    for t in range(m // _MT):
        tok = pl.ds(t * _MT, _MT)
        r = res_ref[tok, :]
        a2 = jnp.sum(r * r, axis=1, keepdims=True)

        def _scan_chunk(c, carry):
            mval, midx = carry
            cbc = cb_ref[0, pl.ds(c * _CK, _CK), :]
            ab = jax.lax.dot_general(r, cbc, (((1,), (1,)), ((), ())),
                                     preferred_element_type=jnp.float32)
            b2 = jnp.sum(cbc * cbc, axis=1)
            d2 = (a2 - 2.0 * ab) + b2[None, :]
            cmin = jnp.min(d2, axis=1, keepdims=True)
            iota = jax.lax.broadcasted_iota(jnp.int32, (_MT, _CK), 1) + c * _CK
            cand = jnp.where(d2 == cmin, iota, k)
            cidx = jnp.min(cand, axis=1, keepdims=True)
            better = cmin < mval
            return (jnp.where(better, cmin, mval),
                    jnp.where(better, cidx, midx))

        mval0 = jnp.full((_MT, 1), jnp.inf, jnp.float32)
        midx0 = jnp.zeros((_MT, 1), jnp.int32)
        _, midx = jax.lax.fori_loop(0, nck, _scan_chunk, (mval0, midx0))

        def _gather_chunk(c, acc):
            cbc = cb_ref[0, pl.ds(c * _CK, _CK), :]
            iota = jax.lax.broadcasted_iota(jnp.int32, (_MT, _CK), 1) + c * _CK
            onehot = (iota == midx).astype(jnp.float32)
            return acc + jax.lax.dot_general(
                onehot, cbc, (((1,), (0,)), ((), ())),
                preferred_element_type=jnp.float32,
                precision=jax.lax.Precision.HIGHEST)

        step = jax.lax.fori_loop(0, nck, _gather_chunk,
                                 jnp.zeros((_MT, d), jnp.float32))

        quant_ref[tok, :] = quant_ref[tok, :] + step
        res_ref[tok, :] = r - step
        x_t = x_ref[tok, :]
        loss_ref[0, 0] += jnp.sum((step - x_t) ** 2) / (m * d)
        idx_ref[0, 0, tok] = midx[:, 0]


@jax.jit
def kernel(audio_features, semantic_context, W_in, b_in, codebooks, W_sem,
           b_sem):
    B, S, D_in = audio_features.shape
    N, K, d = codebooks.shape
    M = B * S
    audio2d = audio_features.reshape(M, D_in)
    b_in2d = b_in.reshape(1, d)
    b_sem3d = b_sem.reshape(b_sem.shape[0], 1, d)

    quant, idx, loss = pl.pallas_call(
        _rvq_body,
        grid=(N,),
        in_specs=[
            pl.BlockSpec((M, D_in), lambda i: (0, 0)),
            pl.BlockSpec(semantic_context.shape, lambda i: (0, 0)),
            pl.BlockSpec((D_in, d), lambda i: (0, 0)),
            pl.BlockSpec((1, d), lambda i: (0, 0)),
            pl.BlockSpec((1, K, d), lambda i: (i, 0, 0)),
            pl.BlockSpec(W_sem.shape, lambda i: (0, 0, 0)),
            pl.BlockSpec(b_sem3d.shape, lambda i: (0, 0, 0)),
        ],
        out_specs=[
            pl.BlockSpec((M, d), lambda i: (0, 0)),
            pl.BlockSpec((1, 1, M), lambda i: (i, 0, 0)),
            pl.BlockSpec(memory_space=pltpu.SMEM),
        ],
        out_shape=[
            jax.ShapeDtypeStruct((M, d), jnp.float32),
            jax.ShapeDtypeStruct((N, 1, M), jnp.int32),
            jax.ShapeDtypeStruct((1, 1), jnp.float32),
        ],
        scratch_shapes=[
            pltpu.VMEM((M, d), jnp.float32),
            pltpu.VMEM((M, d), jnp.float32),
        ],
        compiler_params=pltpu.CompilerParams(
            dimension_semantics=("arbitrary",)),
    )(audio2d, semantic_context, W_in, b_in2d, codebooks, W_sem, b_sem3d)

    return (quant.reshape(B, S, d), idx.reshape(N, B, S), loss[0, 0] * 0.25)


# speedup vs baseline: 1.4013x; 1.4013x over previous
import functools

import jax
import jax.numpy as jnp
from jax.experimental import pallas as pl
from jax.experimental.pallas import tpu as pltpu

_MT = 1024
_CK = 1024


def _rvq_body(audio_ref, sem_ref, w_in_ref, b_in_ref, cb_ref, w_sem_ref,
              b_sem_ref, quant_ref, idx_ref, loss_ref, x_ref, res_ref):
    i = pl.program_id(0)
    m = audio_ref.shape[0]
    d = w_in_ref.shape[1]
    k = cb_ref.shape[1]
    b = sem_ref.shape[0]
    s = m // b
    n_sem = w_sem_ref.shape[0]

    @pl.when(i == 0)
    def _init():
        x = jnp.dot(audio_ref[...], w_in_ref[...],
                    preferred_element_type=jnp.float32) + b_in_ref[...]
        x_ref[...] = x
        res_ref[...] = x
        quant_ref[...] = jnp.zeros_like(quant_ref)
        loss_ref[0, 0] = 0.0

    @pl.when(i < n_sem)
    def _semantic_bias():
        j = jnp.minimum(i, n_sem - 1)
        bias = jnp.dot(sem_ref[...], w_sem_ref[j],
                       preferred_element_type=jnp.float32) + b_sem_ref[j]
        rows = jax.lax.broadcasted_iota(jnp.int32, (m, b), 0) // s
        cols = jax.lax.broadcasted_iota(jnp.int32, (m, b), 1)
        sel = (rows == cols).astype(jnp.float32)
        res_ref[...] = res_ref[...] + 0.1 * jnp.dot(
            sel, bias, preferred_element_type=jnp.float32,
            precision=jax.lax.Precision.HIGHEST)

    nck = k // _CK
    for t in range(m // _MT):
        tok = pl.ds(t * _MT, _MT)
        r = res_ref[tok, :]
        a2 = jnp.sum(r * r, axis=1, keepdims=True)
        r2x = r + r

        def _scan_chunk(c, carry):
            mval, midx = carry
            cbc = cb_ref[0, pl.ds(c * _CK, _CK), :]
            ab2 = jax.lax.dot_general(r2x, cbc, (((1,), (1,)), ((), ())),
                                      preferred_element_type=jnp.float32)
            b2 = jnp.sum(cbc * cbc, axis=1)
            d2 = (a2 - ab2) + b2[None, :]
            cmin = jnp.min(d2, axis=1, keepdims=True)
            iota = jax.lax.broadcasted_iota(jnp.int32, (_MT, _CK), 1) + c * _CK
            cand = jnp.where(d2 == cmin, iota, k)
            cidx = jnp.min(cand, axis=1, keepdims=True)
            better = cmin < mval
            return (jnp.where(better, cmin, mval),
                    jnp.where(better, cidx, midx))

        mval0 = jnp.full((_MT, 1), jnp.inf, jnp.float32)
        midx0 = jnp.zeros((_MT, 1), jnp.int32)
        _, midx = jax.lax.fori_loop(0, nck, _scan_chunk, (mval0, midx0))

        def _gather_chunk(c, acc):
            cbc = cb_ref[0, pl.ds(c * _CK, _CK), :]
            iota = jax.lax.broadcasted_iota(jnp.int32, (_MT, _CK), 1) + c * _CK
            onehot = (iota == midx).astype(jnp.float32)
            hi = cbc.astype(jnp.bfloat16).astype(jnp.float32)
            r1 = cbc - hi
            lo1 = r1.astype(jnp.bfloat16).astype(jnp.float32)
            lo2 = r1 - lo1

            def g(tbl):
                return jax.lax.dot_general(
                    onehot, tbl, (((1,), (0,)), ((), ())),
                    preferred_element_type=jnp.float32)

            return acc + ((g(hi) + g(lo1)) + g(lo2))

        step = jax.lax.fori_loop(0, nck, _gather_chunk,
                                 jnp.zeros((_MT, d), jnp.float32))

        quant_ref[tok, :] = quant_ref[tok, :] + step
        res_ref[tok, :] = r - step
        x_t = x_ref[tok, :]
        loss_ref[0, 0] += jnp.sum((step - x_t) ** 2) / (m * d)
        idx_ref[0, 0, tok] = midx[:, 0]


@jax.jit
def kernel(audio_features, semantic_context, W_in, b_in, codebooks, W_sem,
           b_sem):
    B, S, D_in = audio_features.shape
    N, K, d = codebooks.shape
    M = B * S
    audio2d = audio_features.reshape(M, D_in)
    b_in2d = b_in.reshape(1, d)
    b_sem3d = b_sem.reshape(b_sem.shape[0], 1, d)

    quant, idx, loss = pl.pallas_call(
        _rvq_body,
        grid=(N,),
        in_specs=[
            pl.BlockSpec((M, D_in), lambda i: (0, 0)),
            pl.BlockSpec(semantic_context.shape, lambda i: (0, 0)),
            pl.BlockSpec((D_in, d), lambda i: (0, 0)),
            pl.BlockSpec((1, d), lambda i: (0, 0)),
            pl.BlockSpec((1, K, d), lambda i: (i, 0, 0)),
            pl.BlockSpec(W_sem.shape, lambda i: (0, 0, 0)),
            pl.BlockSpec(b_sem3d.shape, lambda i: (0, 0, 0)),
        ],
        out_specs=[
            pl.BlockSpec((M, d), lambda i: (0, 0)),
            pl.BlockSpec((1, 1, M), lambda i: (i, 0, 0)),
            pl.BlockSpec(memory_space=pltpu.SMEM),
        ],
        out_shape=[
            jax.ShapeDtypeStruct((M, d), jnp.float32),
            jax.ShapeDtypeStruct((N, 1, M), jnp.int32),
            jax.ShapeDtypeStruct((1, 1), jnp.float32),
        ],
        scratch_shapes=[
            pltpu.VMEM((M, d), jnp.float32),
            pltpu.VMEM((M, d), jnp.float32),
        ],
        compiler_params=pltpu.CompilerParams(
            dimension_semantics=("arbitrary",)),
    )(audio2d, semantic_context, W_in, b_in2d, codebooks, W_sem, b_sem3d)

    return (quant.reshape(B, S, d), idx.reshape(N, B, S), loss[0, 0] * 0.25)


# speedup vs baseline: 1.6974x; 1.2114x over previous
import functools

import jax
import jax.numpy as jnp
from jax.experimental import pallas as pl
from jax.experimental.pallas import tpu as pltpu

_MT = 1024
_CK = 1024


def _rvq_body(audio_ref, sem_ref, w_in_ref, b_in_ref, cb_ref, w_sem_ref,
              b_sem_ref, quant_ref, idx_ref, loss_ref, x_ref, res_ref,
              b2_ref, tbl_ref):
    i = pl.program_id(0)
    m = audio_ref.shape[0]
    d = w_in_ref.shape[1]
    k = cb_ref.shape[1]
    b = sem_ref.shape[0]
    s = m // b
    n_sem = w_sem_ref.shape[0]

    @pl.when(i == 0)
    def _init():
        x = jnp.dot(audio_ref[...], w_in_ref[...],
                    preferred_element_type=jnp.float32) + b_in_ref[...]
        x_ref[...] = x
        res_ref[...] = x
        quant_ref[...] = jnp.zeros_like(quant_ref)
        loss_ref[0, 0] = 0.0

    @pl.when(i < n_sem)
    def _semantic_bias():
        j = jnp.minimum(i, n_sem - 1)
        bias = jnp.dot(sem_ref[...], w_sem_ref[j],
                       preferred_element_type=jnp.float32) + b_sem_ref[j]
        rows = jax.lax.broadcasted_iota(jnp.int32, (m, b), 0) // s
        cols = jax.lax.broadcasted_iota(jnp.int32, (m, b), 1)
        sel = (rows == cols).astype(jnp.float32)
        res_ref[...] = res_ref[...] + 0.1 * jnp.dot(
            sel, bias, preferred_element_type=jnp.float32,
            precision=jax.lax.Precision.HIGHEST)

    nck = k // _CK

    def _prep_chunk(c, _):
        cbc = cb_ref[0, pl.ds(c * _CK, _CK), :]
        b2_ref[0, pl.ds(c * _CK, _CK)] = jnp.sum(cbc * cbc, axis=1)
        hi = cbc.astype(jnp.bfloat16).astype(jnp.float32)
        r1 = cbc - hi
        lo1 = r1.astype(jnp.bfloat16).astype(jnp.float32)
        lo2 = r1 - lo1
        tbl_ref[pl.ds(c * _CK, _CK), :] = jnp.concatenate([hi, lo1, lo2],
                                                          axis=1)
        return 0

    jax.lax.fori_loop(0, nck, _prep_chunk, 0)

    for t in range(m // _MT):
        tok = pl.ds(t * _MT, _MT)
        r = res_ref[tok, :]
        a2 = jnp.sum(r * r, axis=1, keepdims=True)
        r2x = r + r

        def _scan_chunk(c, carry):
            mval, midx = carry
            cbc = cb_ref[0, pl.ds(c * _CK, _CK), :]
            ab2 = jax.lax.dot_general(r2x, cbc, (((1,), (1,)), ((), ())),
                                      preferred_element_type=jnp.float32)
            b2 = b2_ref[0, pl.ds(c * _CK, _CK)]
            d2 = (a2 - ab2) + b2[None, :]
            cmin = jnp.min(d2, axis=1, keepdims=True)
            iota = jax.lax.broadcasted_iota(jnp.int32, (_MT, _CK), 1) + c * _CK
            cand = jnp.where(d2 == cmin, iota, k)
            cidx = jnp.min(cand, axis=1, keepdims=True)
            better = cmin < mval
            return (jnp.where(better, cmin, mval),
                    jnp.where(better, cidx, midx))

        mval0 = jnp.full((_MT, 1), jnp.inf, jnp.float32)
        midx0 = jnp.zeros((_MT, 1), jnp.int32)
        _, midx = jax.lax.fori_loop(0, nck, _scan_chunk, (mval0, midx0))

        def _gather_chunk(c, acc):
            tblc = tbl_ref[pl.ds(c * _CK, _CK), :]
            iota = jax.lax.broadcasted_iota(jnp.int32, (_MT, _CK), 1) + c * _CK
            onehot = (iota == midx).astype(jnp.float32)
            return acc + jax.lax.dot_general(
                onehot, tblc, (((1,), (0,)), ((), ())),
                preferred_element_type=jnp.float32)

        step96 = jax.lax.fori_loop(0, nck, _gather_chunk,
                                   jnp.zeros((_MT, 3 * d), jnp.float32))
        step = (step96[:, :d] + step96[:, d:2 * d]) + step96[:, 2 * d:]

        quant_ref[tok, :] = quant_ref[tok, :] + step
        res_ref[tok, :] = r - step
        x_t = x_ref[tok, :]
        loss_ref[0, 0] += jnp.sum((step - x_t) ** 2) / (m * d)
        idx_ref[0, 0, tok] = midx[:, 0]


@jax.jit
def kernel(audio_features, semantic_context, W_in, b_in, codebooks, W_sem,
           b_sem):
    B, S, D_in = audio_features.shape
    N, K, d = codebooks.shape
    M = B * S
    audio2d = audio_features.reshape(M, D_in)
    b_in2d = b_in.reshape(1, d)
    b_sem3d = b_sem.reshape(b_sem.shape[0], 1, d)

    quant, idx, loss = pl.pallas_call(
        _rvq_body,
        grid=(N,),
        in_specs=[
            pl.BlockSpec((M, D_in), lambda i: (0, 0)),
            pl.BlockSpec(semantic_context.shape, lambda i: (0, 0)),
            pl.BlockSpec((D_in, d), lambda i: (0, 0)),
            pl.BlockSpec((1, d), lambda i: (0, 0)),
            pl.BlockSpec((1, K, d), lambda i: (i, 0, 0)),
            pl.BlockSpec(W_sem.shape, lambda i: (0, 0, 0)),
            pl.BlockSpec(b_sem3d.shape, lambda i: (0, 0, 0)),
        ],
        out_specs=[
            pl.BlockSpec((M, d), lambda i: (0, 0)),
            pl.BlockSpec((1, 1, M), lambda i: (i, 0, 0)),
            pl.BlockSpec(memory_space=pltpu.SMEM),
        ],
        out_shape=[
            jax.ShapeDtypeStruct((M, d), jnp.float32),
            jax.ShapeDtypeStruct((N, 1, M), jnp.int32),
            jax.ShapeDtypeStruct((1, 1), jnp.float32),
        ],
        scratch_shapes=[
            pltpu.VMEM((M, d), jnp.float32),
            pltpu.VMEM((M, d), jnp.float32),
            pltpu.VMEM((1, K), jnp.float32),
            pltpu.VMEM((K, 3 * d), jnp.float32),
        ],
        compiler_params=pltpu.CompilerParams(
            dimension_semantics=("arbitrary",)),
    )(audio2d, semantic_context, W_in, b_in2d, codebooks, W_sem, b_sem3d)

    return (quant.reshape(B, S, d), idx.reshape(N, B, S), loss[0, 0] * 0.25)


# speedup vs baseline: 1.8926x; 1.1150x over previous
import functools

import jax
import jax.numpy as jnp
from jax.experimental import pallas as pl
from jax.experimental.pallas import tpu as pltpu

_MT = 1024
_CK = 2048


def _rvq_body(audio_ref, sem_ref, w_in_ref, b_in_ref, cb_ref, w_sem_ref,
              b_sem_ref, quant_ref, idx_ref, loss_ref, x_ref, res_ref,
              b2_ref, tbl_ref):
    i = pl.program_id(0)
    m = audio_ref.shape[0]
    d = w_in_ref.shape[1]
    k = cb_ref.shape[1]
    b = sem_ref.shape[0]
    s = m // b
    n_sem = w_sem_ref.shape[0]

    @pl.when(i == 0)
    def _init():
        x = jnp.dot(audio_ref[...], w_in_ref[...],
                    preferred_element_type=jnp.float32) + b_in_ref[...]
        x_ref[...] = x
        res_ref[...] = x
        quant_ref[...] = jnp.zeros_like(quant_ref)
        loss_ref[0, 0] = 0.0

    @pl.when(i < n_sem)
    def _semantic_bias():
        j = jnp.minimum(i, n_sem - 1)
        bias = jnp.dot(sem_ref[...], w_sem_ref[j],
                       preferred_element_type=jnp.float32) + b_sem_ref[j]
        rows = jax.lax.broadcasted_iota(jnp.int32, (m, b), 0) // s
        cols = jax.lax.broadcasted_iota(jnp.int32, (m, b), 1)
        sel = (rows == cols).astype(jnp.float32)
        res_ref[...] = res_ref[...] + 0.1 * jnp.dot(
            sel, bias, preferred_element_type=jnp.float32,
            precision=jax.lax.Precision.HIGHEST)

    nck = k // _CK

    def _prep_chunk(c, _):
        cbc = cb_ref[0, pl.ds(c * _CK, _CK), :]
        b2_ref[0, pl.ds(c * _CK, _CK)] = jnp.sum(cbc * cbc, axis=1)
        hi = cbc.astype(jnp.bfloat16).astype(jnp.float32)
        r1 = cbc - hi
        lo1 = r1.astype(jnp.bfloat16).astype(jnp.float32)
        lo2 = r1 - lo1
        tbl_ref[pl.ds(c * _CK, _CK), :] = jnp.concatenate([hi, lo1, lo2],
                                                          axis=1)
        return 0

    jax.lax.fori_loop(0, nck, _prep_chunk, 0)

    for t in range(m // _MT):
        tok = pl.ds(t * _MT, _MT)
        r = res_ref[tok, :]
        a2 = jnp.sum(r * r, axis=1, keepdims=True)
        r2x = r + r

        def _scan_chunk(c, carry):
            mval, midx = carry
            cbc = cb_ref[0, pl.ds(c * _CK, _CK), :]
            ab2 = jax.lax.dot_general(r2x, cbc, (((1,), (1,)), ((), ())),
                                      preferred_element_type=jnp.float32)
            b2 = b2_ref[0, pl.ds(c * _CK, _CK)]
            d2 = (a2 - ab2) + b2[None, :]
            cmin = jnp.min(d2, axis=1, keepdims=True)
            iota = jax.lax.broadcasted_iota(jnp.int32, (_MT, _CK), 1) + c * _CK
            cand = jnp.where(d2 == cmin, iota, k)
            cidx = jnp.min(cand, axis=1, keepdims=True)
            better = cmin < mval
            return (jnp.where(better, cmin, mval),
                    jnp.where(better, cidx, midx))

        mval0 = jnp.full((_MT, 1), jnp.inf, jnp.float32)
        midx0 = jnp.zeros((_MT, 1), jnp.int32)
        _, midx = jax.lax.fori_loop(0, nck, _scan_chunk, (mval0, midx0))

        def _gather_chunk(c, acc):
            tblc = tbl_ref[pl.ds(c * _CK, _CK), :]
            iota = jax.lax.broadcasted_iota(jnp.int32, (_MT, _CK), 1) + c * _CK
            onehot = (iota == midx).astype(jnp.float32)
            return acc + jax.lax.dot_general(
                onehot, tblc, (((1,), (0,)), ((), ())),
                preferred_element_type=jnp.float32)

        step96 = jax.lax.fori_loop(0, nck, _gather_chunk,
                                   jnp.zeros((_MT, 3 * d), jnp.float32))
        step = (step96[:, :d] + step96[:, d:2 * d]) + step96[:, 2 * d:]

        quant_ref[tok, :] = quant_ref[tok, :] + step
        res_ref[tok, :] = r - step
        x_t = x_ref[tok, :]
        loss_ref[0, 0] += jnp.sum((step - x_t) ** 2) / (m * d)
        idx_ref[0, 0, tok] = midx[:, 0]


@jax.jit
def kernel(audio_features, semantic_context, W_in, b_in, codebooks, W_sem,
           b_sem):
    B, S, D_in = audio_features.shape
    N, K, d = codebooks.shape
    M = B * S
    audio2d = audio_features.reshape(M, D_in)
    b_in2d = b_in.reshape(1, d)
    b_sem3d = b_sem.reshape(b_sem.shape[0], 1, d)

    quant, idx, loss = pl.pallas_call(
        _rvq_body,
        grid=(N,),
        in_specs=[
            pl.BlockSpec((M, D_in), lambda i: (0, 0)),
            pl.BlockSpec(semantic_context.shape, lambda i: (0, 0)),
            pl.BlockSpec((D_in, d), lambda i: (0, 0)),
            pl.BlockSpec((1, d), lambda i: (0, 0)),
            pl.BlockSpec((1, K, d), lambda i: (i, 0, 0)),
            pl.BlockSpec(W_sem.shape, lambda i: (0, 0, 0)),
            pl.BlockSpec(b_sem3d.shape, lambda i: (0, 0, 0)),
        ],
        out_specs=[
            pl.BlockSpec((M, d), lambda i: (0, 0)),
            pl.BlockSpec((1, 1, M), lambda i: (i, 0, 0)),
            pl.BlockSpec(memory_space=pltpu.SMEM),
        ],
        out_shape=[
            jax.ShapeDtypeStruct((M, d), jnp.float32),
            jax.ShapeDtypeStruct((N, 1, M), jnp.int32),
            jax.ShapeDtypeStruct((1, 1), jnp.float32),
        ],
        scratch_shapes=[
            pltpu.VMEM((M, d), jnp.float32),
            pltpu.VMEM((M, d), jnp.float32),
            pltpu.VMEM((1, K), jnp.float32),
            pltpu.VMEM((K, 3 * d), jnp.float32),
        ],
        compiler_params=pltpu.CompilerParams(
            dimension_semantics=("arbitrary",)),
    )(audio2d, semantic_context, W_in, b_in2d, codebooks, W_sem, b_sem3d)

    return (quant.reshape(B, S, d), idx.reshape(N, B, S), loss[0, 0] * 0.25)


# speedup vs baseline: 1.9651x; 1.0383x over previous
import functools

import jax
import jax.numpy as jnp
from jax.experimental import pallas as pl
from jax.experimental.pallas import tpu as pltpu

_MT = 1024
_CK = 2048


def _rvq_body(audio_ref, sem_ref, w_in_ref, b_in_ref, cb_ref, w_sem_ref,
              b_sem_ref, quant_ref, idx_ref, loss_ref, x_ref, res_ref,
              b2_ref, tbl_ref):
    i = pl.program_id(0)
    m = audio_ref.shape[0]
    d = w_in_ref.shape[1]
    k = cb_ref.shape[1]
    b = sem_ref.shape[0]
    s = m // b
    n_sem = w_sem_ref.shape[0]

    @pl.when(i == 0)
    def _init():
        x = jnp.dot(audio_ref[...], w_in_ref[...],
                    preferred_element_type=jnp.float32) + b_in_ref[...]
        x_ref[...] = x
        res_ref[...] = x
        quant_ref[...] = jnp.zeros_like(quant_ref)
        loss_ref[0, 0] = 0.0

    @pl.when(i < n_sem)
    def _semantic_bias():
        j = jnp.minimum(i, n_sem - 1)
        bias = jnp.dot(sem_ref[...], w_sem_ref[j],
                       preferred_element_type=jnp.float32) + b_sem_ref[j]
        rows = jax.lax.broadcasted_iota(jnp.int32, (m, b), 0) // s
        cols = jax.lax.broadcasted_iota(jnp.int32, (m, b), 1)
        sel = (rows == cols).astype(jnp.float32)
        res_ref[...] = res_ref[...] + 0.1 * jnp.dot(
            sel, bias, preferred_element_type=jnp.float32,
            precision=jax.lax.Precision.HIGHEST)

    nck = k // _CK

    def _prep_chunk(c, _):
        cbc = cb_ref[0, pl.ds(c * _CK, _CK), :]
        b2_ref[0, pl.ds(c * _CK, _CK)] = jnp.sum(cbc * cbc, axis=1)
        hi = cbc.astype(jnp.bfloat16).astype(jnp.float32)
        r1 = cbc - hi
        lo1 = r1.astype(jnp.bfloat16).astype(jnp.float32)
        lo2 = r1 - lo1
        tbl_ref[pl.ds(c * _CK, _CK), :] = jnp.concatenate([hi, lo1, lo2],
                                                          axis=1)
        return 0

    jax.lax.fori_loop(0, nck, _prep_chunk, 0)

    def _scan_ops(c, r2x, a2, mval, midx):
        cbc = cb_ref[0, pl.ds(c * _CK, _CK), :]
        ab2 = jax.lax.dot_general(r2x, cbc, (((1,), (1,)), ((), ())),
                                  preferred_element_type=jnp.float32)
        b2 = b2_ref[0, pl.ds(c * _CK, _CK)]
        d2 = (a2 - ab2) + b2[None, :]
        cmin = jnp.min(d2, axis=1, keepdims=True)
        iota = jax.lax.broadcasted_iota(jnp.int32, (_MT, _CK), 1) + c * _CK
        cand = jnp.where(d2 == cmin, iota, k)
        cidx = jnp.min(cand, axis=1, keepdims=True)
        better = cmin < mval
        return (jnp.where(better, cmin, mval),
                jnp.where(better, cidx, midx))

    def _gather_ops(c, midx_g, acc):
        tblc = tbl_ref[pl.ds(c * _CK, _CK), :]
        iota = jax.lax.broadcasted_iota(jnp.int32, (_MT, _CK), 1) + c * _CK
        onehot = (iota == midx_g).astype(jnp.float32)
        return acc + jax.lax.dot_general(
            onehot, tblc, (((1,), (0,)), ((), ())),
            preferred_element_type=jnp.float32)

    mval0 = jnp.full((_MT, 1), jnp.inf, jnp.float32)
    midx0 = jnp.zeros((_MT, 1), jnp.int32)
    acc0 = jnp.zeros((_MT, 3 * d), jnp.float32)
    ntl = m // _MT

    prev = None
    for t in range(ntl + 1):
        if t < ntl:
            tok = pl.ds(t * _MT, _MT)
            r = res_ref[tok, :]
            a2 = jnp.sum(r * r, axis=1, keepdims=True)
            r2x = r + r
        if t == 0:
            _, midx = jax.lax.fori_loop(
                0, nck, lambda c, cr: _scan_ops(c, r2x, a2, *cr),
                (mval0, midx0))
            acc = None
        elif t < ntl:
            midx_p, r_p = prev

            def _fused(c, cr, _r2x=r2x, _a2=a2, _midx_p=midx_p):
                mv, mi, ac = cr
                mv, mi = _scan_ops(c, _r2x, _a2, mv, mi)
                ac = _gather_ops(c, _midx_p, ac)
                return (mv, mi, ac)

            _, midx, acc = jax.lax.fori_loop(0, nck, _fused,
                                             (mval0, midx0, acc0))
        else:
            midx_p, r_p = prev
            acc = jax.lax.fori_loop(
                0, nck, lambda c, ac: _gather_ops(c, midx_p, ac), acc0)
        if t > 0:
            ptok = pl.ds((t - 1) * _MT, _MT)
            step = (acc[:, :d] + acc[:, d:2 * d]) + acc[:, 2 * d:]
            quant_ref[ptok, :] = quant_ref[ptok, :] + step
            res_ref[ptok, :] = r_p - step
            x_t = x_ref[ptok, :]
            loss_ref[0, 0] += jnp.sum((step - x_t) ** 2) / (m * d)
            idx_ref[0, 0, ptok] = midx_p[:, 0]
        if t < ntl:
            prev = (midx, r)


@jax.jit
def kernel(audio_features, semantic_context, W_in, b_in, codebooks, W_sem,
           b_sem):
    B, S, D_in = audio_features.shape
    N, K, d = codebooks.shape
    M = B * S
    audio2d = audio_features.reshape(M, D_in)
    b_in2d = b_in.reshape(1, d)
    b_sem3d = b_sem.reshape(b_sem.shape[0], 1, d)

    quant, idx, loss = pl.pallas_call(
        _rvq_body,
        grid=(N,),
        in_specs=[
            pl.BlockSpec((M, D_in), lambda i: (0, 0)),
            pl.BlockSpec(semantic_context.shape, lambda i: (0, 0)),
            pl.BlockSpec((D_in, d), lambda i: (0, 0)),
            pl.BlockSpec((1, d), lambda i: (0, 0)),
            pl.BlockSpec((1, K, d), lambda i: (i, 0, 0)),
            pl.BlockSpec(W_sem.shape, lambda i: (0, 0, 0)),
            pl.BlockSpec(b_sem3d.shape, lambda i: (0, 0, 0)),
        ],
        out_specs=[
            pl.BlockSpec((M, d), lambda i: (0, 0)),
            pl.BlockSpec((1, 1, M), lambda i: (i, 0, 0)),
            pl.BlockSpec(memory_space=pltpu.SMEM),
        ],
        out_shape=[
            jax.ShapeDtypeStruct((M, d), jnp.float32),
            jax.ShapeDtypeStruct((N, 1, M), jnp.int32),
            jax.ShapeDtypeStruct((1, 1), jnp.float32),
        ],
        scratch_shapes=[
            pltpu.VMEM((M, d), jnp.float32),
            pltpu.VMEM((M, d), jnp.float32),
            pltpu.VMEM((1, K), jnp.float32),
            pltpu.VMEM((K, 3 * d), jnp.float32),
        ],
        compiler_params=pltpu.CompilerParams(
            dimension_semantics=("arbitrary",)),
    )(audio2d, semantic_context, W_in, b_in2d, codebooks, W_sem, b_sem3d)

    return (quant.reshape(B, S, d), idx.reshape(N, B, S), loss[0, 0] * 0.25)
